# baseline (speedup 1.0000x reference)
import functools

import jax
import jax.numpy as jnp
from jax.experimental import pallas as pl
from jax.experimental.pallas import tpu as pltpu
from jax.experimental.pallas import tpu_sc as plsc

EPS = 1e-6
RSF = 2.5
NEG = -1e30

S, H = 2048, 768
NH, NKV, HD = 12, 4, 64
RD = 32
E, NG = 8, 4
DFF = 512

BS = 256
BQ = 512
BK = 512


def _rope(t, nh, c, s):
    outs = []
    for h in range(nh):
        b = h * HD
        t1 = t[:, b:b + RD // 2]
        t2 = t[:, b + RD // 2:b + RD]
        outs.append(t1 * c - t2 * s)
        outs.append(t2 * c + t1 * s)
        outs.append(t[:, b + RD:b + HD])
    return jnp.concatenate(outs, axis=1)


def _qkv_kernel(x_ref, cos_ref, sin_ref, w1_ref, qwt_ref, kwt_ref, vwt_ref,
                q_ref, k_ref, v_ref):
    x = x_ref[...]
    xn = x * jax.lax.rsqrt(jnp.mean(x * x, -1, keepdims=True) + EPS) * w1_ref[...]
    xnb = xn.astype(jnp.bfloat16)
    q = jnp.dot(xnb, qwt_ref[...], preferred_element_type=jnp.float32)
    k = jnp.dot(xnb, kwt_ref[...], preferred_element_type=jnp.float32)
    v = jnp.dot(xnb, vwt_ref[...], preferred_element_type=jnp.float32)
    c = cos_ref[...][:, :RD // 2]
    s = sin_ref[...][:, :RD // 2]
    q_ref[...] = _rope(q, NH, c, s).astype(jnp.bfloat16)
    k_ref[...] = _rope(k, NKV, c, s).astype(jnp.bfloat16)
    v_ref[...] = v.astype(jnp.bfloat16)


def _attn_kernel(q_ref, k_ref, v_ref, o_ref, s_ref):
    qi = pl.program_id(1)
    qb = q_ref[0]
    rows = qi * BQ + jax.lax.broadcasted_iota(jnp.int32, (BQ, BK), 0)

    def b1(j, m):
        kj = k_ref[0, pl.ds(j * BK, BK), :]
        s = jax.lax.dot_general(qb, kj, (((1,), (1,)), ((), ())),
                                preferred_element_type=jnp.float32) * (HD ** -0.5)
        cols = j * BK + jax.lax.broadcasted_iota(jnp.int32, (BQ, BK), 1)
        s = jnp.where(cols > rows, NEG, s)
        s_ref[:, pl.ds(j * BK, BK)] = s
        return jnp.maximum(m, jnp.max(s, -1, keepdims=True))

    nk = (qi + 1) * (BQ // BK)
    m = jax.lax.fori_loop(0, nk, b1, jnp.full((BQ, 1), NEG, jnp.float32))

    def b2(j, carry):
        l, acc = carry
        s = s_ref[:, pl.ds(j * BK, BK)]
        p = jnp.exp(s - m)
        vj = v_ref[0, pl.ds(j * BK, BK), :]
        acc = acc + jnp.dot(p.astype(jnp.bfloat16), vj,
                            preferred_element_type=jnp.float32)
        return l + jnp.sum(p, -1, keepdims=True), acc

    l, acc = jax.lax.fori_loop(0, nk, b2, (jnp.zeros((BQ, 1), jnp.float32),
                                           jnp.zeros((BQ, HD), jnp.float32)))
    o_ref[0] = acc / l


def _post_kernel(ao_ref, res_ref, owt_ref, w2_ref, gwt_ref,
                 res2_ref, h2_ref, sc_ref):
    h = res_ref[...] + jnp.dot(ao_ref[...], owt_ref[...],
                               preferred_element_type=jnp.float32)
    res2_ref[...] = h
    h2 = h * jax.lax.rsqrt(jnp.mean(h * h, -1, keepdims=True) + EPS) * w2_ref[...]
    h2_ref[...] = h2.astype(jnp.bfloat16)
    sc_ref[...] = jax.nn.sigmoid(jnp.dot(h2, gwt_ref[...],
                                         preferred_element_type=jnp.float32))


def _top1_mask(vals):
    m = jnp.max(vals, -1, keepdims=True)
    eq = vals == m
    k = vals.shape[-1]
    io = jax.lax.broadcasted_iota(jnp.int32, vals.shape, 1)
    first = jnp.min(jnp.where(eq, io, k), -1, keepdims=True)
    return io == first


BLK = 256
CAP = 4096 + 8 * BLK
NBLK = CAP // BLK


def _route_kernel(sc_ref, gb_ref, w_ref, p_ref, blk_ref):
    sc = sc_ref[...]
    sfc = sc + gb_ref[...]
    g = jnp.concatenate(
        [sfc[:, 2 * j:2 * j + 1] + sfc[:, 2 * j + 1:2 * j + 2]
         for j in range(NG)], axis=1)
    g1 = _top1_mask(g)
    g2 = _top1_mask(jnp.where(g1, NEG, g))
    gm = jnp.where(g1 | g2, 1.0, 0.0)
    smask = jnp.concatenate([gm[:, j // 2:j // 2 + 1] for j in range(E)],
                            axis=1)
    tmp = jnp.where(smask > 0.5, sfc, NEG)
    e1 = _top1_mask(tmp)
    e2 = _top1_mask(jnp.where(e1, NEG, tmp))
    tw1 = jnp.sum(jnp.where(e1, sc, 0.0), -1, keepdims=True)
    tw2 = jnp.sum(jnp.where(e2, sc, 0.0), -1, keepdims=True)
    denom = tw1 + tw2 + 1e-20
    w_ref[...] = jnp.concatenate([tw1, tw2], axis=1) / denom * RSF
    e1f = jnp.where(e1, 1.0, 0.0)
    e2f = jnp.where(e2, 1.0, 0.0)
    m = e1f + e2f
    tri = (jax.lax.broadcasted_iota(jnp.int32, (BS, BS), 0) >
           jax.lax.broadcasted_iota(jnp.int32, (BS, BS), 1)).astype(jnp.float32)
    ranks = []
    carry = jnp.zeros((1, E), jnp.float32)
    for c in range(S // BS):
        mc = m[c * BS:(c + 1) * BS]
        ranks.append(jax.lax.dot_general(
            tri, mc, (((1,), (0,)), ((), ())),
            precision=jax.lax.Precision.HIGHEST,
            preferred_element_type=jnp.float32) + carry)
        carry = carry + jnp.sum(mc, 0, keepdims=True)
    rank = jnp.concatenate(ranks, axis=0)
    counts = carry
    rc = jnp.floor((counts + (BLK - 1)) / BLK) * BLK
    ends = jnp.concatenate(
        [jnp.sum(rc[:, :j + 1], axis=1, keepdims=True) for j in range(E)],
        axis=1)
    starts = ends - rc
    pos = starts + rank
    p0 = jnp.sum(e1f * pos, axis=1, keepdims=True)
    p1 = jnp.sum(e2f * pos, axis=1, keepdims=True)
    p_ref[...] = jnp.concatenate([p0, p1], axis=1).astype(jnp.int32)
    bI = (jax.lax.broadcasted_iota(jnp.int32, (1, NBLK), 1) * BLK
          ).astype(jnp.float32)
    acc = jnp.zeros((1, NBLK), jnp.int32)
    for e in range(E):
        acc = acc + jnp.where(bI >= ends[:, e:e + 1], 1, 0)
    blk_ref[...] = acc


def _moe_kernel(blk_ref, xg_ref, wg_ref, wu_ref, wd_ref, yg_ref):
    xb = xg_ref[...]
    hg = jnp.dot(xb, wg_ref[0], preferred_element_type=jnp.float32)
    hu = jnp.dot(xb, wu_ref[0], preferred_element_type=jnp.float32)
    act = (hg * jax.lax.logistic(hg) * hu).astype(jnp.bfloat16)
    yg_ref[...] = jnp.dot(act, wd_ref[0], preferred_element_type=jnp.float32)


def _combine_kernel(res_ref, y0_ref, y1_ref, w_ref, out_ref):
    w = w_ref[...]
    out_ref[...] = (res_ref[...] + w[:, 0:1] * y0_ref[...]
                    + w[:, 1:2] * y1_ref[...])


def _sc_dispatch(xi_hbm, p0_hbm, p1_hbm, xg_hbm, idx_v, rows_v, sem):
    nc, ns = 2, 16
    wid = jax.lax.axis_index("s") * nc + jax.lax.axis_index("c")
    chunk = S // (nc * ns)
    base = wid * chunk
    pltpu.sync_copy(xi_hbm.at[pl.ds(base, chunk)], rows_v)
    pltpu.sync_copy(p0_hbm.at[pl.ds(base, chunk)], idx_v)
    pltpu.async_copy(rows_v, xg_hbm.at[idx_v], sem).wait()
    pltpu.sync_copy(p1_hbm.at[pl.ds(base, chunk)], idx_v)
    pltpu.async_copy(rows_v, xg_hbm.at[idx_v], sem).wait()


def _sc_combine(yg_hbm, p0_hbm, p1_hbm, y0_hbm, y1_hbm, idx_v, rows_v, sem):
    nc, ns = 2, 16
    wid = jax.lax.axis_index("s") * nc + jax.lax.axis_index("c")
    chunk = S // (nc * ns)
    base = wid * chunk
    pltpu.sync_copy(p0_hbm.at[pl.ds(base, chunk)], idx_v)
    pltpu.async_copy(yg_hbm.at[idx_v], rows_v, sem).wait()
    pltpu.sync_copy(rows_v, y0_hbm.at[pl.ds(base, chunk)])
    pltpu.sync_copy(p1_hbm.at[pl.ds(base, chunk)], idx_v)
    pltpu.async_copy(yg_hbm.at[idx_v], rows_v, sem).wait()
    pltpu.sync_copy(rows_v, y1_hbm.at[pl.ds(base, chunk)])


def kernel(hidden_states, cos, sin, ln1_w, ln2_w, q_w, k_w, v_w, o_w,
           gate_w, gate_b, wg, wu, wd):
    x = hidden_states.reshape(S, H)
    cos2 = cos.reshape(S, RD)
    sin2 = sin.reshape(S, RD)

    q, k, v = pl.pallas_call(
        _qkv_kernel,
        grid=(S // BS,),
        in_specs=[
            pl.BlockSpec((BS, H), lambda i: (i, 0)),
            pl.BlockSpec((BS, RD), lambda i: (i, 0)),
            pl.BlockSpec((BS, RD), lambda i: (i, 0)),
            pl.BlockSpec((1, H), lambda i: (0, 0)),
            pl.BlockSpec((H, NH * HD), lambda i: (0, 0)),
            pl.BlockSpec((H, NKV * HD), lambda i: (0, 0)),
            pl.BlockSpec((H, NKV * HD), lambda i: (0, 0)),
        ],
        out_specs=[
            pl.BlockSpec((BS, NH * HD), lambda i: (i, 0)),
            pl.BlockSpec((BS, NKV * HD), lambda i: (i, 0)),
            pl.BlockSpec((BS, NKV * HD), lambda i: (i, 0)),
        ],
        out_shape=[
            jax.ShapeDtypeStruct((S, NH * HD), jnp.bfloat16),
            jax.ShapeDtypeStruct((S, NKV * HD), jnp.bfloat16),
            jax.ShapeDtypeStruct((S, NKV * HD), jnp.bfloat16),
        ],
    )(x, cos2, sin2, ln1_w.reshape(1, H), q_w.T.astype(jnp.bfloat16), k_w.T.astype(jnp.bfloat16), v_w.T.astype(jnp.bfloat16))

    qh = q.reshape(S, NH, HD).transpose(1, 0, 2)
    kh = k.reshape(S, NKV, HD).transpose(1, 0, 2)
    vh = v.reshape(S, NKV, HD).transpose(1, 0, 2)

    rep = NH // NKV
    ao = pl.pallas_call(
        _attn_kernel,
        grid=(NH, S // BQ),
        in_specs=[
            pl.BlockSpec((1, BQ, HD), lambda h, i: (h, i, 0)),
            pl.BlockSpec((1, S, HD), lambda h, i: (h // rep, 0, 0)),
            pl.BlockSpec((1, S, HD), lambda h, i: (h // rep, 0, 0)),
        ],
        out_specs=pl.BlockSpec((1, BQ, HD), lambda h, i: (h, i, 0)),
        out_shape=jax.ShapeDtypeStruct((NH, S, HD), jnp.float32),
        scratch_shapes=[pltpu.VMEM((BQ, S), jnp.float32)],
    )(qh, kh, vh)

    ao2 = ao.transpose(1, 0, 2).reshape(S, NH * HD)

    res2, h2, scores = pl.pallas_call(
        _post_kernel,
        grid=(S // BS,),
        in_specs=[
            pl.BlockSpec((BS, NH * HD), lambda i: (i, 0)),
            pl.BlockSpec((BS, H), lambda i: (i, 0)),
            pl.BlockSpec((NH * HD, H), lambda i: (0, 0)),
            pl.BlockSpec((1, H), lambda i: (0, 0)),
            pl.BlockSpec((H, E), lambda i: (0, 0)),
        ],
        out_specs=[
            pl.BlockSpec((BS, H), lambda i: (i, 0)),
            pl.BlockSpec((BS, H), lambda i: (i, 0)),
            pl.BlockSpec((BS, E), lambda i: (i, 0)),
        ],
        out_shape=[
            jax.ShapeDtypeStruct((S, H), jnp.float32),
            jax.ShapeDtypeStruct((S, H), jnp.bfloat16),
            jax.ShapeDtypeStruct((S, E), jnp.float32),
        ],
    )(ao2.astype(jnp.bfloat16), x, o_w.T.astype(jnp.bfloat16), ln2_w.reshape(1, H), gate_w.T)

    w01, p01, blk_e = pl.pallas_call(
        _route_kernel,
        in_specs=[
            pl.BlockSpec((S, E), lambda: (0, 0)),
            pl.BlockSpec((1, E), lambda: (0, 0)),
        ],
        out_specs=[
            pl.BlockSpec((S, 2), lambda: (0, 0)),
            pl.BlockSpec((S, 2), lambda: (0, 0)),
            pl.BlockSpec((1, NBLK), lambda: (0, 0)),
        ],
        out_shape=[
            jax.ShapeDtypeStruct((S, 2), jnp.float32),
            jax.ShapeDtypeStruct((S, 2), jnp.int32),
            jax.ShapeDtypeStruct((1, NBLK), jnp.int32),
        ],
    )(scores, gate_b.reshape(1, E))

    p0 = p01[:, 0]
    p1 = p01[:, 1]
    xi = jax.lax.bitcast_convert_type(h2.reshape(S, H // 2, 2),
                                      jnp.int32)

    sc_mesh = plsc.VectorSubcoreMesh(core_axis_name="c", subcore_axis_name="s")
    chunk = S // 32
    xg_i = pl.kernel(
        _sc_dispatch,
        mesh=sc_mesh,
        out_type=jax.ShapeDtypeStruct((CAP, H // 2), jnp.int32),
        scratch_types=[
            pltpu.VMEM((chunk,), jnp.int32),
            pltpu.VMEM((chunk, H // 2), jnp.int32),
            pltpu.SemaphoreType.DMA,
        ],
    )(xi, p0, p1)
    xg = jax.lax.bitcast_convert_type(xg_i, jnp.bfloat16).reshape(CAP, H)

    yg = pl.pallas_call(
        _moe_kernel,
        grid_spec=pltpu.PrefetchScalarGridSpec(
            num_scalar_prefetch=1,
            grid=(NBLK,),
            in_specs=[
                pl.BlockSpec((BLK, H), lambda i, b: (i, 0)),
                pl.BlockSpec((1, H, DFF), lambda i, b: (b[0, i], 0, 0)),
                pl.BlockSpec((1, H, DFF), lambda i, b: (b[0, i], 0, 0)),
                pl.BlockSpec((1, DFF, H), lambda i, b: (b[0, i], 0, 0)),
            ],
            out_specs=pl.BlockSpec((BLK, H), lambda i, b: (i, 0)),
        ),
        out_shape=jax.ShapeDtypeStruct((CAP, H), jnp.float32),
    )(blk_e, xg, wg.astype(jnp.bfloat16), wu.astype(jnp.bfloat16),
      wd.astype(jnp.bfloat16))

    y0, y1 = pl.kernel(
        _sc_combine,
        mesh=sc_mesh,
        out_type=(jax.ShapeDtypeStruct((S, H), jnp.float32),
                  jax.ShapeDtypeStruct((S, H), jnp.float32)),
        scratch_types=[
            pltpu.VMEM((chunk,), jnp.int32),
            pltpu.VMEM((chunk, H), jnp.float32),
            pltpu.SemaphoreType.DMA,
        ],
    )(yg, p0, p1)

    out = pl.pallas_call(
        _combine_kernel,
        grid=(S // BS,),
        in_specs=[
            pl.BlockSpec((BS, H), lambda i: (i, 0)),
            pl.BlockSpec((BS, H), lambda i: (i, 0)),
            pl.BlockSpec((BS, H), lambda i: (i, 0)),
            pl.BlockSpec((BS, 2), lambda i: (i, 0)),
        ],
        out_specs=pl.BlockSpec((BS, H), lambda i: (i, 0)),
        out_shape=jax.ShapeDtypeStruct((S, H), jnp.float32),
    )(res2, y0, y1, w01)

    return out.reshape(1, S, H)


# speedup vs baseline: 1.4020x; 1.4020x over previous
import functools

import jax
import jax.numpy as jnp
from jax.experimental import pallas as pl
from jax.experimental.pallas import tpu as pltpu
from jax.experimental.pallas import tpu_sc as plsc

EPS = 1e-6
RSF = 2.5
NEG = -1e30

S, H = 2048, 768
NH, NKV, HD = 12, 4, 64
RD = 32
E, NG = 8, 4
DFF = 512

BS = 256
BQ = 512
BK = 512


def _rope(t, nh, c, s):
    outs = []
    for h in range(nh):
        b = h * HD
        t1 = t[:, b:b + RD // 2]
        t2 = t[:, b + RD // 2:b + RD]
        outs.append(t1 * c - t2 * s)
        outs.append(t2 * c + t1 * s)
        outs.append(t[:, b + RD:b + HD])
    return jnp.concatenate(outs, axis=1)


def _qkv_kernel(x_ref, cos_ref, sin_ref, w1_ref, qwt_ref, kwt_ref, vwt_ref,
                q_ref, k_ref, v_ref):
    x = x_ref[...]
    xn = x * jax.lax.rsqrt(jnp.mean(x * x, -1, keepdims=True) + EPS) * w1_ref[...]
    xnb = xn.astype(jnp.bfloat16)
    q = jnp.dot(xnb, qwt_ref[...], preferred_element_type=jnp.float32)
    k = jnp.dot(xnb, kwt_ref[...], preferred_element_type=jnp.float32)
    v = jnp.dot(xnb, vwt_ref[...], preferred_element_type=jnp.float32)
    c = cos_ref[...][:, :RD // 2]
    s = sin_ref[...][:, :RD // 2]
    q_ref[...] = _rope(q, NH, c, s).astype(jnp.bfloat16)
    k_ref[...] = _rope(k, NKV, c, s).astype(jnp.bfloat16)
    v_ref[...] = v.astype(jnp.bfloat16)


def _attn_kernel(q_ref, k_ref, v_ref, o_ref, s_ref):
    qi = pl.program_id(1)
    qb = q_ref[0]
    rows = qi * BQ + jax.lax.broadcasted_iota(jnp.int32, (BQ, BK), 0)

    def b1(j, m):
        kj = k_ref[0, pl.ds(j * BK, BK), :]
        s = jax.lax.dot_general(qb, kj, (((1,), (1,)), ((), ())),
                                preferred_element_type=jnp.float32) * (HD ** -0.5)
        cols = j * BK + jax.lax.broadcasted_iota(jnp.int32, (BQ, BK), 1)
        s = jnp.where(cols > rows, NEG, s)
        s_ref[:, pl.ds(j * BK, BK)] = s
        return jnp.maximum(m, jnp.max(s, -1, keepdims=True))

    nk = (qi + 1) * (BQ // BK)
    m = jax.lax.fori_loop(0, nk, b1, jnp.full((BQ, 1), NEG, jnp.float32))

    def b2(j, carry):
        l, acc = carry
        s = s_ref[:, pl.ds(j * BK, BK)]
        p = jnp.exp(s - m)
        vj = v_ref[0, pl.ds(j * BK, BK), :]
        acc = acc + jnp.dot(p.astype(jnp.bfloat16), vj,
                            preferred_element_type=jnp.float32)
        return l + jnp.sum(p, -1, keepdims=True), acc

    l, acc = jax.lax.fori_loop(0, nk, b2, (jnp.zeros((BQ, 1), jnp.float32),
                                           jnp.zeros((BQ, HD), jnp.float32)))
    o_ref[0] = acc / l


def _post_kernel(ao_ref, res_ref, owt_ref, w2_ref, gwt_ref,
                 res2_ref, h2_ref, sc_ref):
    h = res_ref[...] + jnp.dot(ao_ref[...], owt_ref[...],
                               preferred_element_type=jnp.float32)
    res2_ref[...] = h
    h2 = h * jax.lax.rsqrt(jnp.mean(h * h, -1, keepdims=True) + EPS) * w2_ref[...]
    h2_ref[...] = h2
    sc_ref[...] = jax.nn.sigmoid(jnp.dot(h2, gwt_ref[...],
                                         preferred_element_type=jnp.float32))


def _top1_mask(vals):
    m = jnp.max(vals, -1, keepdims=True)
    eq = vals == m
    k = vals.shape[-1]
    io = jax.lax.broadcasted_iota(jnp.int32, vals.shape, 1)
    first = jnp.min(jnp.where(eq, io, k), -1, keepdims=True)
    return io == first


BLK = 256
CAP = 4096 + 8 * BLK
NBLK = CAP // BLK


def _route_kernel(sc_ref, gb_ref, w_ref, p_ref, blk_ref):
    sc = sc_ref[...]
    sfc = sc + gb_ref[...]
    g = jnp.concatenate(
        [sfc[:, 2 * j:2 * j + 1] + sfc[:, 2 * j + 1:2 * j + 2]
         for j in range(NG)], axis=1)
    g1 = _top1_mask(g)
    g2 = _top1_mask(jnp.where(g1, NEG, g))
    gm = jnp.where(g1 | g2, 1.0, 0.0)
    smask = jnp.concatenate([gm[:, j // 2:j // 2 + 1] for j in range(E)],
                            axis=1)
    tmp = jnp.where(smask > 0.5, sfc, NEG)
    e1 = _top1_mask(tmp)
    e2 = _top1_mask(jnp.where(e1, NEG, tmp))
    tw1 = jnp.sum(jnp.where(e1, sc, 0.0), -1, keepdims=True)
    tw2 = jnp.sum(jnp.where(e2, sc, 0.0), -1, keepdims=True)
    denom = tw1 + tw2 + 1e-20
    w_ref[...] = jnp.concatenate([tw1, tw2], axis=1) / denom * RSF
    e1f = jnp.where(e1, 1.0, 0.0)
    e2f = jnp.where(e2, 1.0, 0.0)
    m = e1f + e2f
    tri = (jax.lax.broadcasted_iota(jnp.int32, (BS, BS), 0) >
           jax.lax.broadcasted_iota(jnp.int32, (BS, BS), 1)).astype(jnp.float32)
    ranks = []
    carry = jnp.zeros((1, E), jnp.float32)
    for c in range(S // BS):
        mc = m[c * BS:(c + 1) * BS]
        ranks.append(jax.lax.dot_general(
            tri, mc, (((1,), (0,)), ((), ())),
            precision=jax.lax.Precision.HIGHEST,
            preferred_element_type=jnp.float32) + carry)
        carry = carry + jnp.sum(mc, 0, keepdims=True)
    rank = jnp.concatenate(ranks, axis=0)
    counts = carry
    rc = jnp.floor((counts + (BLK - 1)) / BLK) * BLK
    ends = jnp.concatenate(
        [jnp.sum(rc[:, :j + 1], axis=1, keepdims=True) for j in range(E)],
        axis=1)
    starts = ends - rc
    pos = starts + rank
    p0 = jnp.sum(e1f * pos, axis=1, keepdims=True)
    p1 = jnp.sum(e2f * pos, axis=1, keepdims=True)
    p_ref[...] = jnp.concatenate([p0, p1], axis=1).astype(jnp.int32)
    bI = (jax.lax.broadcasted_iota(jnp.int32, (1, NBLK), 1) * BLK
          ).astype(jnp.float32)
    acc = jnp.zeros((1, NBLK), jnp.int32)
    for e in range(E):
        acc = acc + jnp.where(bI >= ends[:, e:e + 1], 1, 0)
    blk_ref[...] = acc


def _moe_kernel(blk_ref, xg_ref, wg_ref, wu_ref, wd_ref, yg_ref):
    xb = xg_ref[...].astype(jnp.bfloat16)
    hg = jnp.dot(xb, wg_ref[0], preferred_element_type=jnp.float32)
    hu = jnp.dot(xb, wu_ref[0], preferred_element_type=jnp.float32)
    act = (hg * jax.lax.logistic(hg) * hu).astype(jnp.bfloat16)
    yg_ref[...] = jnp.dot(act, wd_ref[0], preferred_element_type=jnp.float32)


def _combine_kernel(res_ref, y0_ref, y1_ref, w_ref, out_ref):
    w = w_ref[...]
    out_ref[...] = (res_ref[...] + w[:, 0:1] * y0_ref[...]
                    + w[:, 1:2] * y1_ref[...])


def _sc_dispatch(xi_hbm, p0_hbm, p1_hbm, xg_hbm, idx_v, rows_v, sem):
    nc, ns = 2, 16
    wid = jax.lax.axis_index("s") * nc + jax.lax.axis_index("c")
    chunk = S // (nc * ns)
    base = wid * chunk
    pltpu.sync_copy(xi_hbm.at[pl.ds(base, chunk)], rows_v)
    pltpu.sync_copy(p0_hbm.at[pl.ds(base, chunk)], idx_v)
    pltpu.async_copy(rows_v, xg_hbm.at[idx_v], sem).wait()
    pltpu.sync_copy(p1_hbm.at[pl.ds(base, chunk)], idx_v)
    pltpu.async_copy(rows_v, xg_hbm.at[idx_v], sem).wait()


def _sc_combine(yg_hbm, p0_hbm, p1_hbm, y0_hbm, y1_hbm, idx_v, rows_v, sem):
    nc, ns = 2, 16
    wid = jax.lax.axis_index("s") * nc + jax.lax.axis_index("c")
    chunk = S // (nc * ns)
    base = wid * chunk
    pltpu.sync_copy(p0_hbm.at[pl.ds(base, chunk)], idx_v)
    pltpu.async_copy(yg_hbm.at[idx_v], rows_v, sem).wait()
    pltpu.sync_copy(rows_v, y0_hbm.at[pl.ds(base, chunk)])
    pltpu.sync_copy(p1_hbm.at[pl.ds(base, chunk)], idx_v)
    pltpu.async_copy(yg_hbm.at[idx_v], rows_v, sem).wait()
    pltpu.sync_copy(rows_v, y1_hbm.at[pl.ds(base, chunk)])


def kernel(hidden_states, cos, sin, ln1_w, ln2_w, q_w, k_w, v_w, o_w,
           gate_w, gate_b, wg, wu, wd):
    x = hidden_states.reshape(S, H)
    cos2 = cos.reshape(S, RD)
    sin2 = sin.reshape(S, RD)

    q, k, v = pl.pallas_call(
        _qkv_kernel,
        grid=(S // BS,),
        in_specs=[
            pl.BlockSpec((BS, H), lambda i: (i, 0)),
            pl.BlockSpec((BS, RD), lambda i: (i, 0)),
            pl.BlockSpec((BS, RD), lambda i: (i, 0)),
            pl.BlockSpec((1, H), lambda i: (0, 0)),
            pl.BlockSpec((H, NH * HD), lambda i: (0, 0)),
            pl.BlockSpec((H, NKV * HD), lambda i: (0, 0)),
            pl.BlockSpec((H, NKV * HD), lambda i: (0, 0)),
        ],
        out_specs=[
            pl.BlockSpec((BS, NH * HD), lambda i: (i, 0)),
            pl.BlockSpec((BS, NKV * HD), lambda i: (i, 0)),
            pl.BlockSpec((BS, NKV * HD), lambda i: (i, 0)),
        ],
        out_shape=[
            jax.ShapeDtypeStruct((S, NH * HD), jnp.bfloat16),
            jax.ShapeDtypeStruct((S, NKV * HD), jnp.bfloat16),
            jax.ShapeDtypeStruct((S, NKV * HD), jnp.bfloat16),
        ],
    )(x, cos2, sin2, ln1_w.reshape(1, H), q_w.T.astype(jnp.bfloat16), k_w.T.astype(jnp.bfloat16), v_w.T.astype(jnp.bfloat16))

    qh = q.reshape(S, NH, HD).transpose(1, 0, 2)
    kh = k.reshape(S, NKV, HD).transpose(1, 0, 2)
    vh = v.reshape(S, NKV, HD).transpose(1, 0, 2)

    rep = NH // NKV
    ao = pl.pallas_call(
        _attn_kernel,
        grid=(NH, S // BQ),
        in_specs=[
            pl.BlockSpec((1, BQ, HD), lambda h, i: (h, i, 0)),
            pl.BlockSpec((1, S, HD), lambda h, i: (h // rep, 0, 0)),
            pl.BlockSpec((1, S, HD), lambda h, i: (h // rep, 0, 0)),
        ],
        out_specs=pl.BlockSpec((1, BQ, HD), lambda h, i: (h, i, 0)),
        out_shape=jax.ShapeDtypeStruct((NH, S, HD), jnp.float32),
        scratch_shapes=[pltpu.VMEM((BQ, S), jnp.float32)],
    )(qh, kh, vh)

    ao2 = ao.transpose(1, 0, 2).reshape(S, NH * HD)

    res2, h2, scores = pl.pallas_call(
        _post_kernel,
        grid=(S // BS,),
        in_specs=[
            pl.BlockSpec((BS, NH * HD), lambda i: (i, 0)),
            pl.BlockSpec((BS, H), lambda i: (i, 0)),
            pl.BlockSpec((NH * HD, H), lambda i: (0, 0)),
            pl.BlockSpec((1, H), lambda i: (0, 0)),
            pl.BlockSpec((H, E), lambda i: (0, 0)),
        ],
        out_specs=[
            pl.BlockSpec((BS, H), lambda i: (i, 0)),
            pl.BlockSpec((BS, H), lambda i: (i, 0)),
            pl.BlockSpec((BS, E), lambda i: (i, 0)),
        ],
        out_shape=[
            jax.ShapeDtypeStruct((S, H), jnp.float32),
            jax.ShapeDtypeStruct((S, H), jnp.float32),
            jax.ShapeDtypeStruct((S, E), jnp.float32),
        ],
    )(ao2.astype(jnp.bfloat16), x, o_w.T.astype(jnp.bfloat16), ln2_w.reshape(1, H), gate_w.T)

    w01, p01, blk_e = pl.pallas_call(
        _route_kernel,
        in_specs=[
            pl.BlockSpec((S, E), lambda: (0, 0)),
            pl.BlockSpec((1, E), lambda: (0, 0)),
        ],
        out_specs=[
            pl.BlockSpec((S, 2), lambda: (0, 0)),
            pl.BlockSpec((S, 2), lambda: (0, 0)),
            pl.BlockSpec((1, NBLK), lambda: (0, 0)),
        ],
        out_shape=[
            jax.ShapeDtypeStruct((S, 2), jnp.float32),
            jax.ShapeDtypeStruct((S, 2), jnp.int32),
            jax.ShapeDtypeStruct((1, NBLK), jnp.int32),
        ],
    )(scores, gate_b.reshape(1, E))

    p0 = p01[:, 0]
    p1 = p01[:, 1]

    sc_mesh = plsc.VectorSubcoreMesh(core_axis_name="c", subcore_axis_name="s")
    chunk = S // 32
    xg = pl.kernel(
        _sc_dispatch,
        mesh=sc_mesh,
        out_type=jax.ShapeDtypeStruct((CAP, H), jnp.float32),
        scratch_types=[
            pltpu.VMEM((chunk,), jnp.int32),
            pltpu.VMEM((chunk, H), jnp.float32),
            pltpu.SemaphoreType.DMA,
        ],
    )(h2, p0, p1)

    yg = pl.pallas_call(
        _moe_kernel,
        grid_spec=pltpu.PrefetchScalarGridSpec(
            num_scalar_prefetch=1,
            grid=(NBLK,),
            in_specs=[
                pl.BlockSpec((BLK, H), lambda i, b: (i, 0)),
                pl.BlockSpec((1, H, DFF), lambda i, b: (b[0, i], 0, 0)),
                pl.BlockSpec((1, H, DFF), lambda i, b: (b[0, i], 0, 0)),
                pl.BlockSpec((1, DFF, H), lambda i, b: (b[0, i], 0, 0)),
            ],
            out_specs=pl.BlockSpec((BLK, H), lambda i, b: (i, 0)),
        ),
        out_shape=jax.ShapeDtypeStruct((CAP, H), jnp.float32),
    )(blk_e, xg, wg.astype(jnp.bfloat16), wu.astype(jnp.bfloat16),
      wd.astype(jnp.bfloat16))

    y0, y1 = pl.kernel(
        _sc_combine,
        mesh=sc_mesh,
        out_type=(jax.ShapeDtypeStruct((S, H), jnp.float32),
                  jax.ShapeDtypeStruct((S, H), jnp.float32)),
        scratch_types=[
            pltpu.VMEM((chunk,), jnp.int32),
            pltpu.VMEM((chunk, H), jnp.float32),
            pltpu.SemaphoreType.DMA,
        ],
    )(yg, p0, p1)

    out = pl.pallas_call(
        _combine_kernel,
        grid=(S // BS,),
        in_specs=[
            pl.BlockSpec((BS, H), lambda i: (i, 0)),
            pl.BlockSpec((BS, H), lambda i: (i, 0)),
            pl.BlockSpec((BS, H), lambda i: (i, 0)),
            pl.BlockSpec((BS, 2), lambda i: (i, 0)),
        ],
        out_specs=pl.BlockSpec((BS, H), lambda i: (i, 0)),
        out_shape=jax.ShapeDtypeStruct((S, H), jnp.float32),
    )(res2, y0, y1, w01)

    return out.reshape(1, S, H)


# speedup vs baseline: 1.6028x; 1.1433x over previous
import functools

import jax
import jax.numpy as jnp
from jax.experimental import pallas as pl
from jax.experimental.pallas import tpu as pltpu
from jax.experimental.pallas import tpu_sc as plsc

EPS = 1e-6
RSF = 2.5
NEG = -1e30

S, H = 2048, 768
NH, NKV, HD = 12, 4, 64
RD = 32
E, NG = 8, 4
DFF = 512

BS = 256
BQ = 512
BK = 512


def _rope(t, nh, c, s):
    outs = []
    for h in range(nh):
        b = h * HD
        t1 = t[:, b:b + RD // 2]
        t2 = t[:, b + RD // 2:b + RD]
        outs.append(t1 * c - t2 * s)
        outs.append(t2 * c + t1 * s)
        outs.append(t[:, b + RD:b + HD])
    return jnp.concatenate(outs, axis=1)


def _qkv_kernel(x_ref, cos_ref, sin_ref, w1_ref, qwt_ref, kwt_ref, vwt_ref,
                q_ref, k_ref, v_ref):
    x = x_ref[...]
    xn = x * jax.lax.rsqrt(jnp.mean(x * x, -1, keepdims=True) + EPS) * w1_ref[...]
    xnb = xn.astype(jnp.bfloat16)
    q = jnp.dot(xnb, qwt_ref[...], preferred_element_type=jnp.float32)
    k = jnp.dot(xnb, kwt_ref[...], preferred_element_type=jnp.float32)
    v = jnp.dot(xnb, vwt_ref[...], preferred_element_type=jnp.float32)
    c = cos_ref[...][:, :RD // 2]
    s = sin_ref[...][:, :RD // 2]
    q_ref[...] = _rope(q, NH, c, s).astype(jnp.bfloat16)
    k_ref[...] = _rope(k, NKV, c, s).astype(jnp.bfloat16)
    v_ref[...] = v.astype(jnp.bfloat16)


def _attn_kernel(q_ref, k_ref, vT_ref, o_ref, s_ref):
    qi = pl.program_id(1)
    qb = q_ref[0]
    lanes_q = qi * BQ + jax.lax.broadcasted_iota(jnp.int32, (BK, BQ), 1)

    def b1(j, m):
        kj = k_ref[0, pl.ds(j * BK, BK), :]
        sT = jax.lax.dot_general(kj, qb, (((1,), (1,)), ((), ())),
                                 preferred_element_type=jnp.float32) * (HD ** -0.5)
        subs_k = j * BK + jax.lax.broadcasted_iota(jnp.int32, (BK, BQ), 0)
        sT = jnp.where(subs_k > lanes_q, NEG, sT)
        s_ref[pl.ds(j * BK, BK), :] = sT
        return jnp.maximum(m, jnp.max(sT, 0, keepdims=True))

    nk = (qi + 1) * (BQ // BK)
    m = jax.lax.fori_loop(0, nk, b1, jnp.full((1, BQ), NEG, jnp.float32))

    def b2(j, carry):
        l, acc = carry
        sT = s_ref[pl.ds(j * BK, BK), :]
        pT = jnp.exp(sT - m)
        vj = vT_ref[0, :, pl.ds(j * BK, BK)]
        acc = acc + jax.lax.dot_general(vj, pT.astype(jnp.bfloat16),
                                        (((1,), (0,)), ((), ())),
                                        preferred_element_type=jnp.float32)
        return l + jnp.sum(pT, 0, keepdims=True), acc

    l, acc = jax.lax.fori_loop(0, nk, b2, (jnp.zeros((1, BQ), jnp.float32),
                                           jnp.zeros((HD, BQ), jnp.float32)))
    o_ref[0] = acc / l


def _post_kernel(ao_ref, res_ref, owt_ref, w2_ref, gwt_ref,
                 res2_ref, h2_ref, sc_ref):
    h = res_ref[...] + jnp.dot(ao_ref[...], owt_ref[...],
                               preferred_element_type=jnp.float32)
    res2_ref[...] = h
    h2 = h * jax.lax.rsqrt(jnp.mean(h * h, -1, keepdims=True) + EPS) * w2_ref[...]
    h2_ref[...] = h2
    sc_ref[...] = jax.nn.sigmoid(jnp.dot(h2, gwt_ref[...],
                                         preferred_element_type=jnp.float32))


def _top1_mask(vals):
    m = jnp.max(vals, -1, keepdims=True)
    eq = vals == m
    k = vals.shape[-1]
    io = jax.lax.broadcasted_iota(jnp.int32, vals.shape, 1)
    first = jnp.min(jnp.where(eq, io, k), -1, keepdims=True)
    return io == first


BLK = 256
CAP = 4096 + 8 * BLK
NBLK = CAP // BLK


def _route_kernel(sc_ref, gb_ref, w_ref, p_ref, blk_ref):
    sc = sc_ref[...]
    sfc = sc + gb_ref[...]
    g = jnp.concatenate(
        [sfc[:, 2 * j:2 * j + 1] + sfc[:, 2 * j + 1:2 * j + 2]
         for j in range(NG)], axis=1)
    g1 = _top1_mask(g)
    g2 = _top1_mask(jnp.where(g1, NEG, g))
    gm = jnp.where(g1 | g2, 1.0, 0.0)
    smask = jnp.concatenate([gm[:, j // 2:j // 2 + 1] for j in range(E)],
                            axis=1)
    tmp = jnp.where(smask > 0.5, sfc, NEG)
    e1 = _top1_mask(tmp)
    e2 = _top1_mask(jnp.where(e1, NEG, tmp))
    tw1 = jnp.sum(jnp.where(e1, sc, 0.0), -1, keepdims=True)
    tw2 = jnp.sum(jnp.where(e2, sc, 0.0), -1, keepdims=True)
    denom = tw1 + tw2 + 1e-20
    w_ref[...] = jnp.concatenate([tw1, tw2], axis=1) / denom * RSF
    e1f = jnp.where(e1, 1.0, 0.0)
    e2f = jnp.where(e2, 1.0, 0.0)
    m = e1f + e2f
    tri = (jax.lax.broadcasted_iota(jnp.int32, (BS, BS), 0) >
           jax.lax.broadcasted_iota(jnp.int32, (BS, BS), 1)).astype(jnp.float32)
    ranks = []
    carry = jnp.zeros((1, E), jnp.float32)
    for c in range(S // BS):
        mc = m[c * BS:(c + 1) * BS]
        ranks.append(jax.lax.dot_general(
            tri, mc, (((1,), (0,)), ((), ())),
            precision=jax.lax.Precision.HIGHEST,
            preferred_element_type=jnp.float32) + carry)
        carry = carry + jnp.sum(mc, 0, keepdims=True)
    rank = jnp.concatenate(ranks, axis=0)
    counts = carry
    rc = jnp.floor((counts + (BLK - 1)) / BLK) * BLK
    ends = jnp.concatenate(
        [jnp.sum(rc[:, :j + 1], axis=1, keepdims=True) for j in range(E)],
        axis=1)
    starts = ends - rc
    pos = starts + rank
    p0 = jnp.sum(e1f * pos, axis=1, keepdims=True)
    p1 = jnp.sum(e2f * pos, axis=1, keepdims=True)
    p_ref[...] = jnp.concatenate([p0, p1], axis=1).astype(jnp.int32)
    bI = (jax.lax.broadcasted_iota(jnp.int32, (1, NBLK), 1) * BLK
          ).astype(jnp.float32)
    acc = jnp.zeros((1, NBLK), jnp.int32)
    for e in range(E):
        acc = acc + jnp.where(bI >= ends[:, e:e + 1], 1, 0)
    blk_ref[...] = acc


def _moe_kernel(blk_ref, xg_ref, wg_ref, wu_ref, wd_ref, yg_ref):
    xb = xg_ref[...].astype(jnp.bfloat16)
    hg = jnp.dot(xb, wg_ref[0], preferred_element_type=jnp.float32)
    hu = jnp.dot(xb, wu_ref[0], preferred_element_type=jnp.float32)
    act = (hg * jax.lax.logistic(hg) * hu).astype(jnp.bfloat16)
    yg_ref[...] = jnp.dot(act, wd_ref[0], preferred_element_type=jnp.float32)


def _combine_kernel(res_ref, y0_ref, y1_ref, w_ref, out_ref):
    w = w_ref[...]
    out_ref[...] = (res_ref[...] + w[:, 0:1] * y0_ref[...]
                    + w[:, 1:2] * y1_ref[...])


def _sc_dispatch(xi_hbm, p0_hbm, p1_hbm, xg_hbm, idx_v, rows_v, sem):
    nc, ns = 2, 16
    wid = jax.lax.axis_index("s") * nc + jax.lax.axis_index("c")
    chunk = S // (nc * ns)
    base = wid * chunk
    pltpu.sync_copy(xi_hbm.at[pl.ds(base, chunk)], rows_v)
    pltpu.sync_copy(p0_hbm.at[pl.ds(base, chunk)], idx_v)
    pltpu.async_copy(rows_v, xg_hbm.at[idx_v], sem).wait()
    pltpu.sync_copy(p1_hbm.at[pl.ds(base, chunk)], idx_v)
    pltpu.async_copy(rows_v, xg_hbm.at[idx_v], sem).wait()


def _sc_combine(yg_hbm, p0_hbm, p1_hbm, y0_hbm, y1_hbm, idx_v, rows_v, sem):
    nc, ns = 2, 16
    wid = jax.lax.axis_index("s") * nc + jax.lax.axis_index("c")
    chunk = S // (nc * ns)
    base = wid * chunk
    pltpu.sync_copy(p0_hbm.at[pl.ds(base, chunk)], idx_v)
    pltpu.async_copy(yg_hbm.at[idx_v], rows_v, sem).wait()
    pltpu.sync_copy(rows_v, y0_hbm.at[pl.ds(base, chunk)])
    pltpu.sync_copy(p1_hbm.at[pl.ds(base, chunk)], idx_v)
    pltpu.async_copy(yg_hbm.at[idx_v], rows_v, sem).wait()
    pltpu.sync_copy(rows_v, y1_hbm.at[pl.ds(base, chunk)])


def kernel(hidden_states, cos, sin, ln1_w, ln2_w, q_w, k_w, v_w, o_w,
           gate_w, gate_b, wg, wu, wd):
    x = hidden_states.reshape(S, H)
    cos2 = cos.reshape(S, RD)
    sin2 = sin.reshape(S, RD)

    q, k, v = pl.pallas_call(
        _qkv_kernel,
        grid=(S // BS,),
        in_specs=[
            pl.BlockSpec((BS, H), lambda i: (i, 0)),
            pl.BlockSpec((BS, RD), lambda i: (i, 0)),
            pl.BlockSpec((BS, RD), lambda i: (i, 0)),
            pl.BlockSpec((1, H), lambda i: (0, 0)),
            pl.BlockSpec((H, NH * HD), lambda i: (0, 0)),
            pl.BlockSpec((H, NKV * HD), lambda i: (0, 0)),
            pl.BlockSpec((H, NKV * HD), lambda i: (0, 0)),
        ],
        out_specs=[
            pl.BlockSpec((BS, NH * HD), lambda i: (i, 0)),
            pl.BlockSpec((BS, NKV * HD), lambda i: (i, 0)),
            pl.BlockSpec((BS, NKV * HD), lambda i: (i, 0)),
        ],
        out_shape=[
            jax.ShapeDtypeStruct((S, NH * HD), jnp.bfloat16),
            jax.ShapeDtypeStruct((S, NKV * HD), jnp.bfloat16),
            jax.ShapeDtypeStruct((S, NKV * HD), jnp.bfloat16),
        ],
    )(x, cos2, sin2, ln1_w.reshape(1, H), q_w.T.astype(jnp.bfloat16), k_w.T.astype(jnp.bfloat16), v_w.T.astype(jnp.bfloat16))

    qh = q.reshape(S, NH, HD).transpose(1, 0, 2)
    kh = k.reshape(S, NKV, HD).transpose(1, 0, 2)
    vh = v.reshape(S, NKV, HD).transpose(1, 2, 0)

    rep = NH // NKV
    ao = pl.pallas_call(
        _attn_kernel,
        grid=(NH, S // BQ),
        in_specs=[
            pl.BlockSpec((1, BQ, HD), lambda h, i: (h, i, 0)),
            pl.BlockSpec((1, S, HD), lambda h, i: (h // rep, 0, 0)),
            pl.BlockSpec((1, HD, S), lambda h, i: (h // rep, 0, 0)),
        ],
        out_specs=pl.BlockSpec((1, HD, BQ), lambda h, i: (h, 0, i)),
        out_shape=jax.ShapeDtypeStruct((NH, HD, S), jnp.float32),
        scratch_shapes=[pltpu.VMEM((S, BQ), jnp.float32)],
    )(qh, kh, vh)

    ao2 = ao.transpose(2, 0, 1).reshape(S, NH * HD)

    res2, h2, scores = pl.pallas_call(
        _post_kernel,
        grid=(S // BS,),
        in_specs=[
            pl.BlockSpec((BS, NH * HD), lambda i: (i, 0)),
            pl.BlockSpec((BS, H), lambda i: (i, 0)),
            pl.BlockSpec((NH * HD, H), lambda i: (0, 0)),
            pl.BlockSpec((1, H), lambda i: (0, 0)),
            pl.BlockSpec((H, E), lambda i: (0, 0)),
        ],
        out_specs=[
            pl.BlockSpec((BS, H), lambda i: (i, 0)),
            pl.BlockSpec((BS, H), lambda i: (i, 0)),
            pl.BlockSpec((BS, E), lambda i: (i, 0)),
        ],
        out_shape=[
            jax.ShapeDtypeStruct((S, H), jnp.float32),
            jax.ShapeDtypeStruct((S, H), jnp.float32),
            jax.ShapeDtypeStruct((S, E), jnp.float32),
        ],
    )(ao2.astype(jnp.bfloat16), x, o_w.T.astype(jnp.bfloat16), ln2_w.reshape(1, H), gate_w.T)

    w01, p01, blk_e = pl.pallas_call(
        _route_kernel,
        in_specs=[
            pl.BlockSpec((S, E), lambda: (0, 0)),
            pl.BlockSpec((1, E), lambda: (0, 0)),
        ],
        out_specs=[
            pl.BlockSpec((S, 2), lambda: (0, 0)),
            pl.BlockSpec((S, 2), lambda: (0, 0)),
            pl.BlockSpec((1, NBLK), lambda: (0, 0)),
        ],
        out_shape=[
            jax.ShapeDtypeStruct((S, 2), jnp.float32),
            jax.ShapeDtypeStruct((S, 2), jnp.int32),
            jax.ShapeDtypeStruct((1, NBLK), jnp.int32),
        ],
    )(scores, gate_b.reshape(1, E))

    p0 = p01[:, 0]
    p1 = p01[:, 1]

    sc_mesh = plsc.VectorSubcoreMesh(core_axis_name="c", subcore_axis_name="s")
    chunk = S // 32
    xg = pl.kernel(
        _sc_dispatch,
        mesh=sc_mesh,
        out_type=jax.ShapeDtypeStruct((CAP, H), jnp.float32),
        scratch_types=[
            pltpu.VMEM((chunk,), jnp.int32),
            pltpu.VMEM((chunk, H), jnp.float32),
            pltpu.SemaphoreType.DMA,
        ],
    )(h2, p0, p1)

    yg = pl.pallas_call(
        _moe_kernel,
        grid_spec=pltpu.PrefetchScalarGridSpec(
            num_scalar_prefetch=1,
            grid=(NBLK,),
            in_specs=[
                pl.BlockSpec((BLK, H), lambda i, b: (i, 0)),
                pl.BlockSpec((1, H, DFF), lambda i, b: (b[0, i], 0, 0)),
                pl.BlockSpec((1, H, DFF), lambda i, b: (b[0, i], 0, 0)),
                pl.BlockSpec((1, DFF, H), lambda i, b: (b[0, i], 0, 0)),
            ],
            out_specs=pl.BlockSpec((BLK, H), lambda i, b: (i, 0)),
        ),
        out_shape=jax.ShapeDtypeStruct((CAP, H), jnp.float32),
    )(blk_e, xg, wg.astype(jnp.bfloat16), wu.astype(jnp.bfloat16),
      wd.astype(jnp.bfloat16))

    y0, y1 = pl.kernel(
        _sc_combine,
        mesh=sc_mesh,
        out_type=(jax.ShapeDtypeStruct((S, H), jnp.float32),
                  jax.ShapeDtypeStruct((S, H), jnp.float32)),
        scratch_types=[
            pltpu.VMEM((chunk,), jnp.int32),
            pltpu.VMEM((chunk, H), jnp.float32),
            pltpu.SemaphoreType.DMA,
        ],
    )(yg, p0, p1)

    out = pl.pallas_call(
        _combine_kernel,
        grid=(S // BS,),
        in_specs=[
            pl.BlockSpec((BS, H), lambda i: (i, 0)),
            pl.BlockSpec((BS, H), lambda i: (i, 0)),
            pl.BlockSpec((BS, H), lambda i: (i, 0)),
            pl.BlockSpec((BS, 2), lambda i: (i, 0)),
        ],
        out_specs=pl.BlockSpec((BS, H), lambda i: (i, 0)),
        out_shape=jax.ShapeDtypeStruct((S, H), jnp.float32),
    )(res2, y0, y1, w01)

    return out.reshape(1, S, H)
